# baseline (speedup 1.0000x reference)
import functools

import jax
import jax.numpy as jnp
from jax import lax
from jax.experimental import pallas as pl
from jax.experimental.pallas import tpu as pltpu
from jax.experimental.pallas import tpu_sc as plsc

_T, _B, _S, _D = 4, 1, 2048, 1024
_N = _B * _S
_E = 8
_TOPK = 2
_NC = 64
_DFF = 4096
_EF = _DFF // _E
_CPE = _NC // _E
_BETA = 0.5
_THRESH = 1.0

_TM = 256
_G = (_TOPK * _N) // _TM + _E
_P = _G * _TM

_NW = 32
_CHUNK = _N // _NW
_HALF = _CHUNK // 2


def _routing_kernel(x_ref, bias_ref, i1_ref, i2_ref, w1_ref, w2_ref,
                    p1_ref, p2_ref, cnt_ref, rps_ref, base_ref):
    i = pl.program_id(0)
    tn = x_ref.shape[1]
    d = x_ref.shape[2]

    @pl.when(i == 0)
    def _():
        base_ref[...] = jnp.zeros_like(base_ref)

    v = jnp.zeros((tn, d), jnp.float32)
    fr = jnp.zeros((tn, d), jnp.float32)
    for t in range(_T):
        v = _BETA * v + x_ref[t]
        s = (v >= _THRESH).astype(jnp.float32)
        fr = fr + s
        v = v - s * _THRESH
    dmask = ((jnp.right_shift(lax.broadcasted_iota(jnp.int32, (d, _E), 0), 3)
              & (_E - 1))
             == lax.broadcasted_iota(jnp.int32, (d, _E), 1)).astype(jnp.float32)
    es = lax.dot_general(fr, dmask, (((1,), (0,)), ((), ())),
                         preferred_element_type=jnp.float32)
    es = es * (1.0 / (_T * (_D // _NC) * _CPE))
    es = es + bias_ref[0][None, :]
    eidx = lax.broadcasted_iota(jnp.int32, (tn, _E), 1)
    m1 = jnp.max(es, axis=1)
    i1 = jnp.min(jnp.where(es == m1[:, None], eidx, _E), axis=1)
    masked = jnp.where(eidx == i1[:, None], -jnp.inf, es)
    m2 = jnp.max(masked, axis=1)
    i2 = jnp.min(jnp.where(masked == m2[:, None], eidx, _E), axis=1)
    eb = jnp.exp(m2 - m1)
    w1 = 1.0 / (1.0 + eb)
    w2 = eb / (1.0 + eb)
    oh1 = (eidx == i1[:, None]).astype(jnp.float32)
    oh2 = (eidx == i2[:, None]).astype(jnp.float32)
    assigned = oh1 + oh2
    r_iota = lax.broadcasted_iota(jnp.int32, (tn, tn), 0)
    c_iota = lax.broadcasted_iota(jnp.int32, (tn, tn), 1)
    tril = (r_iota > c_iota).astype(jnp.float32)
    ranks = lax.dot_general(tril, assigned, (((1,), (0,)), ((), ())),
                            preferred_element_type=jnp.float32)
    base = base_ref[0][None, :]
    pos = ranks + base
    p1 = jnp.sum(oh1 * pos, axis=1)
    p2 = jnp.sum(oh2 * pos, axis=1)
    base_ref[...] = base + jnp.sum(assigned, axis=0, keepdims=True)

    i1_ref[...] = i1[None, :]
    i2_ref[...] = i2[None, :]
    w1_ref[...] = jnp.broadcast_to(w1[:, None], (tn, 128))
    w2_ref[...] = jnp.broadcast_to(w2[:, None], (tn, 128))
    p1_ref[...] = p1[None, :].astype(jnp.int32)
    p2_ref[...] = p2[None, :].astype(jnp.int32)

    cnt = jnp.sum(assigned, axis=0)
    ex = jnp.exp(es - m1[:, None])
    rp = ex / jnp.sum(ex, axis=1, keepdims=True)
    rps = jnp.sum(rp, axis=0)

    @pl.when(i == 0)
    def _():
        cnt_ref[...] = cnt[None, :]
        rps_ref[...] = rps[None, :]

    @pl.when(i > 0)
    def _():
        cnt_ref[...] += cnt[None, :]
        rps_ref[...] += rps[None, :]


def _slots_kernel(i1_ref, i2_ref, p1_ref, p2_ref, cnt_ref, s1_ref, s2_ref):
    i1v = i1_ref[...]
    i2v = i2_ref[...]
    off1 = jnp.zeros_like(i1v)
    off2 = jnp.zeros_like(i2v)
    running = 0
    for e in range(_E):
        off1 = jnp.where(i1v == e, running, off1)
        off2 = jnp.where(i2v == e, running, off2)
        ce = cnt_ref[0, e]
        running = running + ((ce + (_TM - 1)) // _TM) * _TM
    s1_ref[...] = p1_ref[...] + off1
    s2_ref[...] = p2_ref[...] + off2


_SUB = 32


def _dispatch_body(x_hbm, s1_hbm, s2_hbm, w1_hbm, w2_hbm,
                   xg0, xg1, xg2, xg3, ws_hbm,
                   s1_v, s2_v, s1h, s2h, wrow_v, xa, xb,
                   semw, sia, sib, ssa1, ssa2, ssb1, ssb2):
    c = lax.axis_index("c")
    s = lax.axis_index("s")
    wid = s * 2 + c
    base = wid * _CHUNK
    pltpu.sync_copy(s1_hbm.at[pl.ds(base, _CHUNK)], s1_v)
    pltpu.sync_copy(s2_hbm.at[pl.ds(base, _CHUNK)], s2_v)
    for hh in range(_CHUNK // _SUB):
        pltpu.sync_copy(s1_hbm.at[pl.ds(base + hh * _SUB, _SUB)], s1h.at[hh])
        pltpu.sync_copy(s2_hbm.at[pl.ds(base + hh * _SUB, _SUB)], s2h.at[hh])
    pltpu.sync_copy(w1_hbm.at[pl.ds(base, _CHUNK)], wrow_v)
    cpw = pltpu.async_copy(wrow_v, ws_hbm.at[s1_v], semw)
    cpw.wait()
    pltpu.sync_copy(w2_hbm.at[pl.ds(base, _CHUNK)], wrow_v)
    cpw = pltpu.async_copy(wrow_v, ws_hbm.at[s2_v], semw)
    cpw.wait()
    xgs = [xg0, xg1, xg2, xg3]
    bufs = [xa, xb]
    isems = [sia, sib]
    ssems = [(ssa1, ssa2), (ssb1, ssb2)]
    nh = _CHUNK // _SUB
    nstep = _T * nh
    ins = [None, None]
    scs = [None, None]

    def issue_in(k):
        par = k & 1
        t, hh = divmod(k, nh)
        ins[par] = pltpu.async_copy(
            x_hbm.at[pl.ds(t * _N + base + hh * _SUB, _SUB)],
            bufs[par], isems[par])

    issue_in(0)
    for k in range(nstep):
        par = k & 1
        t, hh = divmod(k, nh)
        if k + 1 < nstep:
            if scs[1 - par] is not None:
                scs[1 - par][0].wait()
                scs[1 - par][1].wait()
                scs[1 - par] = None
            issue_in(k + 1)
        ins[par].wait()
        c1 = pltpu.async_copy(bufs[par], xgs[t].at[s1h.at[hh]],
                              ssems[par][0])
        c2 = pltpu.async_copy(bufs[par], xgs[t].at[s2h.at[hh]],
                              ssems[par][1])
        scs[par] = (c1, c2)
    for par in range(2):
        if scs[par] is not None:
            scs[par][0].wait()
            scs[par][1].wait()


def _make_dispatch():
    mesh = plsc.VectorSubcoreMesh(core_axis_name="c", subcore_axis_name="s")
    out_type = (
        [jax.ShapeDtypeStruct((_P, _D), jnp.float32) for _ in range(_T)]
        + [jax.ShapeDtypeStruct((_P, 128), jnp.float32)]
    )
    scratch = [
        pltpu.VMEM((_CHUNK,), jnp.int32),
        pltpu.VMEM((_CHUNK,), jnp.int32),
        pltpu.VMEM((_CHUNK // _SUB, _SUB), jnp.int32),
        pltpu.VMEM((_CHUNK // _SUB, _SUB), jnp.int32),
        pltpu.VMEM((_CHUNK, 128), jnp.float32),
        pltpu.VMEM((_SUB, _D), jnp.float32),
        pltpu.VMEM((_SUB, _D), jnp.float32),
        pltpu.SemaphoreType.DMA,
        pltpu.SemaphoreType.DMA,
        pltpu.SemaphoreType.DMA,
        pltpu.SemaphoreType.DMA,
        pltpu.SemaphoreType.DMA,
        pltpu.SemaphoreType.DMA,
        pltpu.SemaphoreType.DMA,
    ]
    return pl.kernel(_dispatch_body, mesh=mesh, out_type=out_type,
                     scratch_types=scratch)


def _group_kernel(eid_ref, xg0_ref, xg1_ref, xg2_ref, xg3_ref,
                  ws_ref, wup_ref, wdn_ref,
                  og0_ref, og1_ref, og2_ref, og3_ref):
    g = pl.program_id(0)
    ntiles = eid_ref[_G]

    @pl.when(g < ntiles)
    def _():
        xg_refs = [xg0_ref, xg1_ref, xg2_ref, xg3_ref]
        og_refs = [og0_ref, og1_ref, og2_ref, og3_ref]
        tm = xg0_ref.shape[0]
        wup = wup_ref[0]
        wdn = wdn_ref[0]
        v = jnp.zeros((tm, _EF), jnp.float32)
        h = []
        for t in range(_T):
            u = lax.dot_general(xg_refs[t][...], wup,
                                (((1,), (1,)), ((), ())),
                                preferred_element_type=jnp.float32)
            v = _BETA * v + u
            sp = (v >= _THRESH).astype(jnp.float32)
            h.append(sp)
            v = v - sp * _THRESH
        wcol = ws_ref[:, :1]
        v2 = jnp.zeros((tm, _D), jnp.float32)
        for t in range(_T):
            o = lax.dot_general(h[t], wdn, (((1,), (1,)), ((), ())),
                                preferred_element_type=jnp.float32)
            v2 = _BETA * v2 + o
            s2 = (v2 >= _THRESH).astype(jnp.float32)
            v2 = v2 - s2 * _THRESH
            og_refs[t][...] = s2 * wcol


_QR = 16


def _combine_body(og0, og1, og2, og3, s1_hbm, s2_hbm, out_hbm,
                  s1_v, s2_v, ga1, ga2, gb1, gb2,
                  sg1a, sg2a, sg1b, sg2b, soa, sob):
    c = lax.axis_index("c")
    s = lax.axis_index("s")
    wid = s * 2 + c
    base = wid * _CHUNK
    ogs = [og0, og1, og2, og3]
    pltpu.sync_copy(s1_hbm.at[pl.ds(base, _CHUNK)], s1_v)
    pltpu.sync_copy(s2_hbm.at[pl.ds(base, _CHUNK)], s2_v)
    bufs = [(ga1, ga2), (gb1, gb2)]
    gsems = [(sg1a, sg2a), (sg1b, sg2b)]
    osems = [soa, sob]
    nq = _CHUNK // _QR
    nstep = _T * nq
    gcp = [None, None]
    ocp = [None, None]

    def issue(k):
        par = k & 1
        t, q = divmod(k, nq)
        idx1 = s1_v.at[pl.ds(q * _QR, _QR)]
        idx2 = s2_v.at[pl.ds(q * _QR, _QR)]
        c1 = pltpu.async_copy(ogs[t].at[idx1], bufs[par][0], gsems[par][0])
        c2 = pltpu.async_copy(ogs[t].at[idx2], bufs[par][1], gsems[par][1])
        gcp[par] = (c1, c2)

    issue(0)
    for k in range(nstep):
        par = k & 1
        t, q = divmod(k, nq)
        if k + 1 < nstep:
            if ocp[1 - par] is not None:
                ocp[1 - par].wait()
                ocp[1 - par] = None
            issue(k + 1)
        gcp[par][0].wait()
        gcp[par][1].wait()
        g1, g2 = bufs[par]

        def _addrow(r, _):
            for cc in range(_D // 16):
                sl = pl.ds(cc * 16, 16)
                g1[r, sl] = g1[r, sl] + g2[r, sl]
            return 0

        lax.fori_loop(0, _QR, _addrow, 0)
        ocp[par] = pltpu.async_copy(
            g1, out_hbm.at[pl.ds(t * _N + base + q * _QR, _QR)], osems[par])
    for par in range(2):
        if ocp[par] is not None:
            ocp[par].wait()


def _make_combine():
    mesh = plsc.VectorSubcoreMesh(core_axis_name="c", subcore_axis_name="s")
    out_type = jax.ShapeDtypeStruct((_T * _N, _D), jnp.float32)
    scratch = [
        pltpu.VMEM((_CHUNK,), jnp.int32),
        pltpu.VMEM((_CHUNK,), jnp.int32),
        pltpu.VMEM((_QR, _D), jnp.float32),
        pltpu.VMEM((_QR, _D), jnp.float32),
        pltpu.VMEM((_QR, _D), jnp.float32),
        pltpu.VMEM((_QR, _D), jnp.float32),
        pltpu.SemaphoreType.DMA,
        pltpu.SemaphoreType.DMA,
        pltpu.SemaphoreType.DMA,
        pltpu.SemaphoreType.DMA,
        pltpu.SemaphoreType.DMA,
        pltpu.SemaphoreType.DMA,
    ]
    return pl.kernel(_combine_body, mesh=mesh, out_type=out_type,
                     scratch_types=scratch)


def kernel(x, W_up, W_down, expert_bias):
    Tt, Bb, Ss, Dd = x.shape
    N = Bb * Ss
    xf = x.reshape(Tt, N, Dd)
    bias2d = expert_bias.reshape(1, _E)

    TN = 512
    (i1, i2, wrep1, wrep2, p1, p2, cnt, rps) = pl.pallas_call(
        _routing_kernel,
        grid=(N // TN,),
        in_specs=[
            pl.BlockSpec((Tt, TN, Dd), lambda i: (0, i, 0)),
            pl.BlockSpec((1, _E), lambda i: (0, 0)),
        ],
        out_specs=[
            pl.BlockSpec((1, TN), lambda i: (0, i)),
            pl.BlockSpec((1, TN), lambda i: (0, i)),
            pl.BlockSpec((TN, 128), lambda i: (i, 0)),
            pl.BlockSpec((TN, 128), lambda i: (i, 0)),
            pl.BlockSpec((1, TN), lambda i: (0, i)),
            pl.BlockSpec((1, TN), lambda i: (0, i)),
            pl.BlockSpec((1, _E), lambda i: (0, 0)),
            pl.BlockSpec((1, _E), lambda i: (0, 0)),
        ],
        out_shape=[
            jax.ShapeDtypeStruct((1, N), jnp.int32),
            jax.ShapeDtypeStruct((1, N), jnp.int32),
            jax.ShapeDtypeStruct((N, 128), jnp.float32),
            jax.ShapeDtypeStruct((N, 128), jnp.float32),
            jax.ShapeDtypeStruct((1, N), jnp.int32),
            jax.ShapeDtypeStruct((1, N), jnp.int32),
            jax.ShapeDtypeStruct((1, _E), jnp.float32),
            jax.ShapeDtypeStruct((1, _E), jnp.float32),
        ],
        scratch_shapes=[pltpu.VMEM((1, _E), jnp.float32)],
    )(xf, bias2d)

    cnt_i = cnt.astype(jnp.int32)
    s1, s2 = pl.pallas_call(
        _slots_kernel,
        grid=(1,),
        in_specs=[
            pl.BlockSpec((1, N), lambda i: (0, 0)),
            pl.BlockSpec((1, N), lambda i: (0, 0)),
            pl.BlockSpec((1, N), lambda i: (0, 0)),
            pl.BlockSpec((1, N), lambda i: (0, 0)),
            pl.BlockSpec(memory_space=pltpu.SMEM),
        ],
        out_specs=[
            pl.BlockSpec((1, N), lambda i: (0, 0)),
            pl.BlockSpec((1, N), lambda i: (0, 0)),
        ],
        out_shape=[
            jax.ShapeDtypeStruct((1, N), jnp.int32),
            jax.ShapeDtypeStruct((1, N), jnp.int32),
        ],
    )(i1, i2, p1, p2, cnt_i)

    tiles = (cnt_i[0] + (_TM - 1)) // _TM
    tcum = jnp.cumsum(tiles)
    g_iota = jnp.arange(_G, dtype=jnp.int32)
    eid = jnp.sum((g_iota[:, None] >= tcum[None, :]).astype(jnp.int32),
                  axis=1)
    eid = jnp.minimum(eid, _E - 1)
    eid = jnp.concatenate([eid, tcum[-1:]])

    dispatch = _make_dispatch()
    s1f = s1.reshape(N)
    s2f = s2.reshape(N)
    xg0, xg1, xg2, xg3, wslot = dispatch(
        xf.reshape(Tt * N, Dd), s1f, s2f, wrep1, wrep2)

    grid_spec = pltpu.PrefetchScalarGridSpec(
        num_scalar_prefetch=1,
        grid=(_G,),
        in_specs=[
            pl.BlockSpec((_TM, Dd), lambda g, eid_ref: (g, 0)),
            pl.BlockSpec((_TM, Dd), lambda g, eid_ref: (g, 0)),
            pl.BlockSpec((_TM, Dd), lambda g, eid_ref: (g, 0)),
            pl.BlockSpec((_TM, Dd), lambda g, eid_ref: (g, 0)),
            pl.BlockSpec((_TM, 128), lambda g, eid_ref: (g, 0)),
            pl.BlockSpec((1, _EF, Dd), lambda g, eid_ref: (eid_ref[g], 0, 0)),
            pl.BlockSpec((1, Dd, _EF), lambda g, eid_ref: (eid_ref[g], 0, 0)),
        ],
        out_specs=[
            pl.BlockSpec((_TM, Dd), lambda g, eid_ref: (g, 0)),
            pl.BlockSpec((_TM, Dd), lambda g, eid_ref: (g, 0)),
            pl.BlockSpec((_TM, Dd), lambda g, eid_ref: (g, 0)),
            pl.BlockSpec((_TM, Dd), lambda g, eid_ref: (g, 0)),
        ],
    )
    og0, og1, og2, og3 = pl.pallas_call(
        _group_kernel,
        grid_spec=grid_spec,
        out_shape=[jax.ShapeDtypeStruct((_P, Dd), jnp.float32)
                   for _ in range(_T)],
    )(eid, xg0, xg1, xg2, xg3, wslot, W_up, W_down)

    combine = _make_combine()
    out_flat = combine(og0, og1, og2, og3, s1f, s2f)

    ef_frac = cnt[0] / (N * _TOPK)
    rp = rps[0] / N
    lb = _E * jnp.sum(ef_frac * rp)
    return out_flat.reshape(Tt, Bb, Ss, Dd), lb


# speedup vs baseline: 1.0537x; 1.0537x over previous
import functools

import jax
import jax.numpy as jnp
from jax import lax
from jax.experimental import pallas as pl
from jax.experimental.pallas import tpu as pltpu
from jax.experimental.pallas import tpu_sc as plsc

_T, _B, _S, _D = 4, 1, 2048, 1024
_N = _B * _S
_E = 8
_TOPK = 2
_NC = 64
_DFF = 4096
_EF = _DFF // _E
_CPE = _NC // _E
_BETA = 0.5
_THRESH = 1.0

_TM = 256
_G = (_TOPK * _N) // _TM + _E
_P = _G * _TM

_NW = 32
_CHUNK = _N // _NW
_HALF = _CHUNK // 2


def _routing_kernel(x_ref, bias_ref, i1_ref, i2_ref, w1_ref, w2_ref,
                    p1_ref, p2_ref, cnt_ref, rps_ref, base_ref):
    i = pl.program_id(0)
    tn = x_ref.shape[1]
    d = x_ref.shape[2]

    @pl.when(i == 0)
    def _():
        base_ref[...] = jnp.zeros_like(base_ref)

    v = jnp.zeros((tn, d), jnp.float32)
    fr = jnp.zeros((tn, d), jnp.float32)
    for t in range(_T):
        v = _BETA * v + x_ref[t]
        s = (v >= _THRESH).astype(jnp.float32)
        fr = fr + s
        v = v - s * _THRESH
    dmask = ((jnp.right_shift(lax.broadcasted_iota(jnp.int32, (d, _E), 0), 3)
              & (_E - 1))
             == lax.broadcasted_iota(jnp.int32, (d, _E), 1)).astype(jnp.float32)
    es = lax.dot_general(fr, dmask, (((1,), (0,)), ((), ())),
                         preferred_element_type=jnp.float32)
    es = es * (1.0 / (_T * (_D // _NC) * _CPE))
    es = es + bias_ref[0][None, :]
    eidx = lax.broadcasted_iota(jnp.int32, (tn, _E), 1)
    m1 = jnp.max(es, axis=1)
    i1 = jnp.min(jnp.where(es == m1[:, None], eidx, _E), axis=1)
    masked = jnp.where(eidx == i1[:, None], -jnp.inf, es)
    m2 = jnp.max(masked, axis=1)
    i2 = jnp.min(jnp.where(masked == m2[:, None], eidx, _E), axis=1)
    eb = jnp.exp(m2 - m1)
    w1 = 1.0 / (1.0 + eb)
    w2 = eb / (1.0 + eb)
    oh1 = (eidx == i1[:, None]).astype(jnp.float32)
    oh2 = (eidx == i2[:, None]).astype(jnp.float32)
    assigned = oh1 + oh2
    r_iota = lax.broadcasted_iota(jnp.int32, (tn, tn), 0)
    c_iota = lax.broadcasted_iota(jnp.int32, (tn, tn), 1)
    tril = (r_iota > c_iota).astype(jnp.float32)
    ranks = lax.dot_general(tril, assigned, (((1,), (0,)), ((), ())),
                            preferred_element_type=jnp.float32)
    base = base_ref[0][None, :]
    pos = ranks + base
    p1 = jnp.sum(oh1 * pos, axis=1)
    p2 = jnp.sum(oh2 * pos, axis=1)
    base_ref[...] = base + jnp.sum(assigned, axis=0, keepdims=True)

    i1_ref[...] = i1[None, :]
    i2_ref[...] = i2[None, :]
    w1_ref[...] = jnp.broadcast_to(w1[:, None], (tn, 128))
    w2_ref[...] = jnp.broadcast_to(w2[:, None], (tn, 128))
    p1_ref[...] = p1[None, :].astype(jnp.int32)
    p2_ref[...] = p2[None, :].astype(jnp.int32)

    cnt = jnp.sum(assigned, axis=0)
    ex = jnp.exp(es - m1[:, None])
    rp = ex / jnp.sum(ex, axis=1, keepdims=True)
    rps = jnp.sum(rp, axis=0)

    @pl.when(i == 0)
    def _():
        cnt_ref[...] = cnt[None, :]
        rps_ref[...] = rps[None, :]

    @pl.when(i > 0)
    def _():
        cnt_ref[...] += cnt[None, :]
        rps_ref[...] += rps[None, :]


def _slots_kernel(i1_ref, i2_ref, p1_ref, p2_ref, cnt_ref, s1_ref, s2_ref):
    i1v = i1_ref[...]
    i2v = i2_ref[...]
    off1 = jnp.zeros_like(i1v)
    off2 = jnp.zeros_like(i2v)
    running = 0
    for e in range(_E):
        off1 = jnp.where(i1v == e, running, off1)
        off2 = jnp.where(i2v == e, running, off2)
        ce = cnt_ref[0, e]
        running = running + ((ce + (_TM - 1)) // _TM) * _TM
    s1_ref[...] = p1_ref[...] + off1
    s2_ref[...] = p2_ref[...] + off2


_SUB = 32


def _dispatch_body(x_hbm, s1_hbm, s2_hbm, w1_hbm, w2_hbm,
                   xg0, xg1, xg2, xg3, ws_hbm,
                   s1_v, s2_v, s1h, s2h, wrow_v, xa, xb,
                   semw, sia, sib, ssa1, ssa2, ssb1, ssb2):
    c = lax.axis_index("c")
    s = lax.axis_index("s")
    wid = s * 2 + c
    base = wid * _CHUNK
    pltpu.sync_copy(s1_hbm.at[pl.ds(base, _CHUNK)], s1_v)
    pltpu.sync_copy(s2_hbm.at[pl.ds(base, _CHUNK)], s2_v)
    for hh in range(_CHUNK // _SUB):
        pltpu.sync_copy(s1_hbm.at[pl.ds(base + hh * _SUB, _SUB)], s1h.at[hh])
        pltpu.sync_copy(s2_hbm.at[pl.ds(base + hh * _SUB, _SUB)], s2h.at[hh])
    pltpu.sync_copy(w1_hbm.at[pl.ds(base, _CHUNK)], wrow_v)
    cpw = pltpu.async_copy(wrow_v, ws_hbm.at[s1_v], semw)
    cpw.wait()
    pltpu.sync_copy(w2_hbm.at[pl.ds(base, _CHUNK)], wrow_v)
    cpw = pltpu.async_copy(wrow_v, ws_hbm.at[s2_v], semw)
    cpw.wait()
    xgs = [xg0, xg1, xg2, xg3]
    bufs = [xa, xb]
    isems = [sia, sib]
    ssems = [(ssa1, ssa2), (ssb1, ssb2)]
    nh = _CHUNK // _SUB
    nstep = _T * nh
    ins = [None, None]
    scs = [None, None]

    def issue_in(k):
        par = k & 1
        t, hh = divmod(k, nh)
        ins[par] = pltpu.async_copy(
            x_hbm.at[pl.ds(t * _N + base + hh * _SUB, _SUB)],
            bufs[par], isems[par])

    issue_in(0)
    for k in range(nstep):
        par = k & 1
        t, hh = divmod(k, nh)
        if k + 1 < nstep:
            if scs[1 - par] is not None:
                scs[1 - par][0].wait()
                scs[1 - par][1].wait()
                scs[1 - par] = None
            issue_in(k + 1)
        ins[par].wait()
        c1 = pltpu.async_copy(bufs[par], xgs[t].at[s1h.at[hh]],
                              ssems[par][0])
        c2 = pltpu.async_copy(bufs[par], xgs[t].at[s2h.at[hh]],
                              ssems[par][1])
        scs[par] = (c1, c2)
    for par in range(2):
        if scs[par] is not None:
            scs[par][0].wait()
            scs[par][1].wait()


def _make_dispatch():
    mesh = plsc.VectorSubcoreMesh(core_axis_name="c", subcore_axis_name="s")
    out_type = (
        [jax.ShapeDtypeStruct((_P, _D), jnp.float32) for _ in range(_T)]
        + [jax.ShapeDtypeStruct((_P, 128), jnp.float32)]
    )
    scratch = [
        pltpu.VMEM((_CHUNK,), jnp.int32),
        pltpu.VMEM((_CHUNK,), jnp.int32),
        pltpu.VMEM((_CHUNK // _SUB, _SUB), jnp.int32),
        pltpu.VMEM((_CHUNK // _SUB, _SUB), jnp.int32),
        pltpu.VMEM((_CHUNK, 128), jnp.float32),
        pltpu.VMEM((_SUB, _D), jnp.float32),
        pltpu.VMEM((_SUB, _D), jnp.float32),
        pltpu.SemaphoreType.DMA,
        pltpu.SemaphoreType.DMA,
        pltpu.SemaphoreType.DMA,
        pltpu.SemaphoreType.DMA,
        pltpu.SemaphoreType.DMA,
        pltpu.SemaphoreType.DMA,
        pltpu.SemaphoreType.DMA,
    ]
    return pl.kernel(_dispatch_body, mesh=mesh, out_type=out_type,
                     scratch_types=scratch)


def _group_kernel(eid_ref, xg0_ref, xg1_ref, xg2_ref, xg3_ref,
                  ws_ref, wup_ref, wdn_ref,
                  og0_ref, og1_ref, og2_ref, og3_ref):
    g = pl.program_id(0)
    ntiles = eid_ref[2 * _G]

    @pl.when(g < ntiles)
    def _():
        xg_refs = [xg0_ref, xg1_ref, xg2_ref, xg3_ref]
        og_refs = [og0_ref, og1_ref, og2_ref, og3_ref]
        tm = xg0_ref.shape[0]
        wup = wup_ref[0]
        wdn = wdn_ref[0]
        v = jnp.zeros((tm, _EF), jnp.float32)
        h = []
        for t in range(_T):
            u = lax.dot_general(xg_refs[t][...], wup,
                                (((1,), (1,)), ((), ())),
                                preferred_element_type=jnp.float32)
            v = _BETA * v + u
            sp = (v >= _THRESH).astype(jnp.float32)
            h.append(sp)
            v = v - sp * _THRESH
        wcol = ws_ref[:, :1]
        v2 = jnp.zeros((tm, _D), jnp.float32)
        for t in range(_T):
            o = lax.dot_general(h[t], wdn, (((1,), (1,)), ((), ())),
                                preferred_element_type=jnp.float32)
            v2 = _BETA * v2 + o
            s2 = (v2 >= _THRESH).astype(jnp.float32)
            v2 = v2 - s2 * _THRESH
            og_refs[t][...] = s2 * wcol


_QR = 16


def _combine_body(og0, og1, og2, og3, s1_hbm, s2_hbm, out_hbm,
                  s1_v, s2_v, ga1, ga2, gb1, gb2,
                  sg1a, sg2a, sg1b, sg2b, soa, sob):
    c = lax.axis_index("c")
    s = lax.axis_index("s")
    wid = s * 2 + c
    base = wid * _CHUNK
    ogs = [og0, og1, og2, og3]
    pltpu.sync_copy(s1_hbm.at[pl.ds(base, _CHUNK)], s1_v)
    pltpu.sync_copy(s2_hbm.at[pl.ds(base, _CHUNK)], s2_v)
    bufs = [(ga1, ga2), (gb1, gb2)]
    gsems = [(sg1a, sg2a), (sg1b, sg2b)]
    osems = [soa, sob]
    nq = _CHUNK // _QR
    nstep = _T * nq
    gcp = [None, None]
    ocp = [None, None]

    def issue(k):
        par = k & 1
        t, q = divmod(k, nq)
        idx1 = s1_v.at[pl.ds(q * _QR, _QR)]
        idx2 = s2_v.at[pl.ds(q * _QR, _QR)]
        c1 = pltpu.async_copy(ogs[t].at[idx1], bufs[par][0], gsems[par][0])
        c2 = pltpu.async_copy(ogs[t].at[idx2], bufs[par][1], gsems[par][1])
        gcp[par] = (c1, c2)

    issue(0)
    for k in range(nstep):
        par = k & 1
        t, q = divmod(k, nq)
        if k + 1 < nstep:
            if ocp[1 - par] is not None:
                ocp[1 - par].wait()
                ocp[1 - par] = None
            issue(k + 1)
        gcp[par][0].wait()
        gcp[par][1].wait()
        g1, g2 = bufs[par]

        def _addrow(r, _):
            for cc in range(_D // 16):
                sl = pl.ds(cc * 16, 16)
                g1[r, sl] = g1[r, sl] + g2[r, sl]
            return 0

        lax.fori_loop(0, _QR, _addrow, 0)
        ocp[par] = pltpu.async_copy(
            g1, out_hbm.at[pl.ds(t * _N + base + q * _QR, _QR)], osems[par])
    for par in range(2):
        if ocp[par] is not None:
            ocp[par].wait()


def _make_combine():
    mesh = plsc.VectorSubcoreMesh(core_axis_name="c", subcore_axis_name="s")
    out_type = jax.ShapeDtypeStruct((_T * _N, _D), jnp.float32)
    scratch = [
        pltpu.VMEM((_CHUNK,), jnp.int32),
        pltpu.VMEM((_CHUNK,), jnp.int32),
        pltpu.VMEM((_QR, _D), jnp.float32),
        pltpu.VMEM((_QR, _D), jnp.float32),
        pltpu.VMEM((_QR, _D), jnp.float32),
        pltpu.VMEM((_QR, _D), jnp.float32),
        pltpu.SemaphoreType.DMA,
        pltpu.SemaphoreType.DMA,
        pltpu.SemaphoreType.DMA,
        pltpu.SemaphoreType.DMA,
        pltpu.SemaphoreType.DMA,
        pltpu.SemaphoreType.DMA,
    ]
    return pl.kernel(_combine_body, mesh=mesh, out_type=out_type,
                     scratch_types=scratch)


def kernel(x, W_up, W_down, expert_bias):
    Tt, Bb, Ss, Dd = x.shape
    N = Bb * Ss
    xf = x.reshape(Tt, N, Dd)
    bias2d = expert_bias.reshape(1, _E)

    TN = 512
    (i1, i2, wrep1, wrep2, p1, p2, cnt, rps) = pl.pallas_call(
        _routing_kernel,
        grid=(N // TN,),
        in_specs=[
            pl.BlockSpec((Tt, TN, Dd), lambda i: (0, i, 0)),
            pl.BlockSpec((1, _E), lambda i: (0, 0)),
        ],
        out_specs=[
            pl.BlockSpec((1, TN), lambda i: (0, i)),
            pl.BlockSpec((1, TN), lambda i: (0, i)),
            pl.BlockSpec((TN, 128), lambda i: (i, 0)),
            pl.BlockSpec((TN, 128), lambda i: (i, 0)),
            pl.BlockSpec((1, TN), lambda i: (0, i)),
            pl.BlockSpec((1, TN), lambda i: (0, i)),
            pl.BlockSpec((1, _E), lambda i: (0, 0)),
            pl.BlockSpec((1, _E), lambda i: (0, 0)),
        ],
        out_shape=[
            jax.ShapeDtypeStruct((1, N), jnp.int32),
            jax.ShapeDtypeStruct((1, N), jnp.int32),
            jax.ShapeDtypeStruct((N, 128), jnp.float32),
            jax.ShapeDtypeStruct((N, 128), jnp.float32),
            jax.ShapeDtypeStruct((1, N), jnp.int32),
            jax.ShapeDtypeStruct((1, N), jnp.int32),
            jax.ShapeDtypeStruct((1, _E), jnp.float32),
            jax.ShapeDtypeStruct((1, _E), jnp.float32),
        ],
        scratch_shapes=[pltpu.VMEM((1, _E), jnp.float32)],
    )(xf, bias2d)

    cnt_i = cnt.astype(jnp.int32)
    s1, s2 = pl.pallas_call(
        _slots_kernel,
        grid=(1,),
        in_specs=[
            pl.BlockSpec((1, N), lambda i: (0, 0)),
            pl.BlockSpec((1, N), lambda i: (0, 0)),
            pl.BlockSpec((1, N), lambda i: (0, 0)),
            pl.BlockSpec((1, N), lambda i: (0, 0)),
            pl.BlockSpec(memory_space=pltpu.SMEM),
        ],
        out_specs=[
            pl.BlockSpec((1, N), lambda i: (0, 0)),
            pl.BlockSpec((1, N), lambda i: (0, 0)),
        ],
        out_shape=[
            jax.ShapeDtypeStruct((1, N), jnp.int32),
            jax.ShapeDtypeStruct((1, N), jnp.int32),
        ],
    )(i1, i2, p1, p2, cnt_i)

    tiles = (cnt_i[0] + (_TM - 1)) // _TM
    tcum = jnp.cumsum(tiles)
    g_iota = jnp.arange(_G, dtype=jnp.int32)
    eid = jnp.sum((g_iota[:, None] >= tcum[None, :]).astype(jnp.int32),
                  axis=1)
    eid = jnp.minimum(eid, _E - 1)
    ntiles = tcum[-1:]
    gmap = jnp.minimum(g_iota, ntiles - 1)
    eid = jnp.concatenate([eid, gmap, ntiles])

    dispatch = _make_dispatch()
    s1f = s1.reshape(N)
    s2f = s2.reshape(N)
    xg0, xg1, xg2, xg3, wslot = dispatch(
        xf.reshape(Tt * N, Dd), s1f, s2f, wrep1, wrep2)

    grid_spec = pltpu.PrefetchScalarGridSpec(
        num_scalar_prefetch=1,
        grid=(_G,),
        in_specs=[
            pl.BlockSpec((_TM, Dd), lambda g, m: (m[_G + g], 0)),
            pl.BlockSpec((_TM, Dd), lambda g, m: (m[_G + g], 0)),
            pl.BlockSpec((_TM, Dd), lambda g, m: (m[_G + g], 0)),
            pl.BlockSpec((_TM, Dd), lambda g, m: (m[_G + g], 0)),
            pl.BlockSpec((_TM, 128), lambda g, m: (m[_G + g], 0)),
            pl.BlockSpec((1, _EF, Dd), lambda g, m: (m[g], 0, 0)),
            pl.BlockSpec((1, Dd, _EF), lambda g, m: (m[g], 0, 0)),
        ],
        out_specs=[
            pl.BlockSpec((_TM, Dd), lambda g, m: (m[_G + g], 0)),
            pl.BlockSpec((_TM, Dd), lambda g, m: (m[_G + g], 0)),
            pl.BlockSpec((_TM, Dd), lambda g, m: (m[_G + g], 0)),
            pl.BlockSpec((_TM, Dd), lambda g, m: (m[_G + g], 0)),
        ],
    )
    og0, og1, og2, og3 = pl.pallas_call(
        _group_kernel,
        grid_spec=grid_spec,
        out_shape=[jax.ShapeDtypeStruct((_P, Dd), jnp.float32)
                   for _ in range(_T)],
    )(eid, xg0, xg1, xg2, xg3, wslot, W_up, W_down)

    combine = _make_combine()
    out_flat = combine(og0, og1, og2, og3, s1f, s2f)

    ef_frac = cnt[0] / (N * _TOPK)
    rp = rps[0] / N
    lb = _E * jnp.sum(ef_frac * rp)
    return out_flat.reshape(Tt, Bb, Ss, Dd), lb


# speedup vs baseline: 1.1205x; 1.0634x over previous
import functools

import jax
import jax.numpy as jnp
from jax import lax
from jax.experimental import pallas as pl
from jax.experimental.pallas import tpu as pltpu
from jax.experimental.pallas import tpu_sc as plsc

_T, _B, _S, _D = 4, 1, 2048, 1024
_N = _B * _S
_E = 8
_TOPK = 2
_NC = 64
_DFF = 4096
_EF = _DFF // _E
_CPE = _NC // _E
_BETA = 0.5
_THRESH = 1.0

_TM = 256
_G = (_TOPK * _N) // _TM + _E
_P = _G * _TM

_NW = 32
_CHUNK = _N // _NW
_HALF = _CHUNK // 2


def _routing_kernel(x_ref, bias_ref, i1_ref, i2_ref, w1_ref, w2_ref,
                    p1_ref, p2_ref, cnt_ref, rps_ref, xbf_ref, base_ref):
    i = pl.program_id(0)
    tn = x_ref.shape[1]
    d = x_ref.shape[2]

    @pl.when(i == 0)
    def _():
        base_ref[...] = jnp.zeros_like(base_ref)

    v = jnp.zeros((tn, d), jnp.float32)
    fr = jnp.zeros((tn, d), jnp.float32)
    for t in range(_T):
        v = _BETA * v + x_ref[t]
        s = (v >= _THRESH).astype(jnp.float32)
        fr = fr + s
        v = v - s * _THRESH
    dmask = ((jnp.right_shift(lax.broadcasted_iota(jnp.int32, (d, _E), 0), 3)
              & (_E - 1))
             == lax.broadcasted_iota(jnp.int32, (d, _E), 1)).astype(jnp.float32)
    es = lax.dot_general(fr, dmask, (((1,), (0,)), ((), ())),
                         preferred_element_type=jnp.float32)
    es = es * (1.0 / (_T * (_D // _NC) * _CPE))
    es = es + bias_ref[0][None, :]
    eidx = lax.broadcasted_iota(jnp.int32, (tn, _E), 1)
    m1 = jnp.max(es, axis=1)
    i1 = jnp.min(jnp.where(es == m1[:, None], eidx, _E), axis=1)
    masked = jnp.where(eidx == i1[:, None], -jnp.inf, es)
    m2 = jnp.max(masked, axis=1)
    i2 = jnp.min(jnp.where(masked == m2[:, None], eidx, _E), axis=1)
    eb = jnp.exp(m2 - m1)
    w1 = 1.0 / (1.0 + eb)
    w2 = eb / (1.0 + eb)
    oh1 = (eidx == i1[:, None]).astype(jnp.float32)
    oh2 = (eidx == i2[:, None]).astype(jnp.float32)
    assigned = oh1 + oh2
    r_iota = lax.broadcasted_iota(jnp.int32, (tn, tn), 0)
    c_iota = lax.broadcasted_iota(jnp.int32, (tn, tn), 1)
    tril = (r_iota > c_iota).astype(jnp.float32)
    ranks = lax.dot_general(tril, assigned, (((1,), (0,)), ((), ())),
                            preferred_element_type=jnp.float32)
    base = base_ref[0][None, :]
    pos = ranks + base
    p1 = jnp.sum(oh1 * pos, axis=1)
    p2 = jnp.sum(oh2 * pos, axis=1)
    base_ref[...] = base + jnp.sum(assigned, axis=0, keepdims=True)

    mask_hi = jnp.int32(-65536)
    xbits = []
    for t in range(_T):
        rt = x_ref[t].astype(jnp.bfloat16).astype(jnp.float32)
        xbits.append(lax.bitcast_convert_type(rt, jnp.int32))
    w01 = (xbits[1] & mask_hi) | lax.shift_right_logical(xbits[0], 16)
    w23 = (xbits[3] & mask_hi) | lax.shift_right_logical(xbits[2], 16)
    xbf_ref[0] = w01
    xbf_ref[1] = w23
    i1_ref[...] = i1[None, :]
    i2_ref[...] = i2[None, :]
    w1_ref[...] = jnp.broadcast_to(w1[:, None], (tn, 128))
    w2_ref[...] = jnp.broadcast_to(w2[:, None], (tn, 128))
    p1_ref[...] = p1[None, :].astype(jnp.int32)
    p2_ref[...] = p2[None, :].astype(jnp.int32)

    cnt = jnp.sum(assigned, axis=0)
    ex = jnp.exp(es - m1[:, None])
    rp = ex / jnp.sum(ex, axis=1, keepdims=True)
    rps = jnp.sum(rp, axis=0)

    @pl.when(i == 0)
    def _():
        cnt_ref[...] = cnt[None, :]
        rps_ref[...] = rps[None, :]

    @pl.when(i > 0)
    def _():
        cnt_ref[...] += cnt[None, :]
        rps_ref[...] += rps[None, :]


def _slots_kernel(i1_ref, i2_ref, p1_ref, p2_ref, cnt_ref, s1_ref, s2_ref):
    i1v = i1_ref[...]
    i2v = i2_ref[...]
    off1 = jnp.zeros_like(i1v)
    off2 = jnp.zeros_like(i2v)
    running = 0
    for e in range(_E):
        off1 = jnp.where(i1v == e, running, off1)
        off2 = jnp.where(i2v == e, running, off2)
        ce = cnt_ref[0, e]
        running = running + ((ce + (_TM - 1)) // _TM) * _TM
    s1_ref[...] = p1_ref[...] + off1
    s2_ref[...] = p2_ref[...] + off2


_SUB = 32


def _dispatch_body(x_hbm, s1_hbm, s2_hbm, w1_hbm, w2_hbm,
                   xg01, xg23, ws_hbm,
                   s1_v, s2_v, s1h, s2h, wrow_v, xa, xb,
                   semw, sia, sib, ssa1, ssa2, ssb1, ssb2):
    c = lax.axis_index("c")
    s = lax.axis_index("s")
    wid = s * 2 + c
    base = wid * _CHUNK
    pltpu.sync_copy(s1_hbm.at[pl.ds(base, _CHUNK)], s1_v)
    pltpu.sync_copy(s2_hbm.at[pl.ds(base, _CHUNK)], s2_v)
    for hh in range(_CHUNK // _SUB):
        pltpu.sync_copy(s1_hbm.at[pl.ds(base + hh * _SUB, _SUB)], s1h.at[hh])
        pltpu.sync_copy(s2_hbm.at[pl.ds(base + hh * _SUB, _SUB)], s2h.at[hh])
    pltpu.sync_copy(w1_hbm.at[pl.ds(base, _CHUNK)], wrow_v)
    cpw = pltpu.async_copy(wrow_v, ws_hbm.at[s1_v], semw)
    cpw.wait()
    pltpu.sync_copy(w2_hbm.at[pl.ds(base, _CHUNK)], wrow_v)
    cpw = pltpu.async_copy(wrow_v, ws_hbm.at[s2_v], semw)
    cpw.wait()
    xgs = [xg01, xg23]
    bufs = [xa, xb]
    isems = [sia, sib]
    ssems = [(ssa1, ssa2), (ssb1, ssb2)]
    nh = _CHUNK // _SUB
    nstep = 2 * nh
    ins = [None, None]
    scs = [None, None]

    def issue_in(k):
        par = k & 1
        t, hh = divmod(k, nh)
        ins[par] = pltpu.async_copy(
            x_hbm.at[pl.ds(t * _N + base + hh * _SUB, _SUB)],
            bufs[par], isems[par])

    issue_in(0)
    for k in range(nstep):
        par = k & 1
        t, hh = divmod(k, nh)
        if k + 1 < nstep:
            if scs[1 - par] is not None:
                scs[1 - par][0].wait()
                scs[1 - par][1].wait()
                scs[1 - par] = None
            issue_in(k + 1)
        ins[par].wait()
        c1 = pltpu.async_copy(bufs[par], xgs[t].at[s1h.at[hh]],
                              ssems[par][0])
        c2 = pltpu.async_copy(bufs[par], xgs[t].at[s2h.at[hh]],
                              ssems[par][1])
        scs[par] = (c1, c2)
    for par in range(2):
        if scs[par] is not None:
            scs[par][0].wait()
            scs[par][1].wait()


def _make_dispatch():
    mesh = plsc.VectorSubcoreMesh(core_axis_name="c", subcore_axis_name="s")
    out_type = (
        [jax.ShapeDtypeStruct((_P, _D), jnp.int32) for _ in range(2)]
        + [jax.ShapeDtypeStruct((_P, 128), jnp.float32)]
    )
    scratch = [
        pltpu.VMEM((_CHUNK,), jnp.int32),
        pltpu.VMEM((_CHUNK,), jnp.int32),
        pltpu.VMEM((_CHUNK // _SUB, _SUB), jnp.int32),
        pltpu.VMEM((_CHUNK // _SUB, _SUB), jnp.int32),
        pltpu.VMEM((_CHUNK, 128), jnp.float32),
        pltpu.VMEM((_SUB, _D), jnp.int32),
        pltpu.VMEM((_SUB, _D), jnp.int32),
        pltpu.SemaphoreType.DMA,
        pltpu.SemaphoreType.DMA,
        pltpu.SemaphoreType.DMA,
        pltpu.SemaphoreType.DMA,
        pltpu.SemaphoreType.DMA,
        pltpu.SemaphoreType.DMA,
        pltpu.SemaphoreType.DMA,
    ]
    return pl.kernel(_dispatch_body, mesh=mesh, out_type=out_type,
                     scratch_types=scratch)


def _group_kernel(eid_ref, xg01_ref, xg23_ref,
                  ws_ref, wup_ref, wdn_ref,
                  og0_ref, og1_ref, og2_ref, og3_ref):
    g = pl.program_id(0)
    ntiles = eid_ref[2 * _G]

    @pl.when(g < ntiles)
    def _():
        og_refs = [og0_ref, og1_ref, og2_ref, og3_ref]
        tm = xg01_ref.shape[0]
        mask_hi = jnp.int32(-65536)
        xs = []
        for wref in (xg01_ref, xg23_ref):
            w = wref[...]
            xs.append(lax.bitcast_convert_type(
                lax.shift_left(w, 16), jnp.float32))
            xs.append(lax.bitcast_convert_type(w & mask_hi, jnp.float32))
        wup = wup_ref[0]
        wdn = wdn_ref[0]
        v = jnp.zeros((tm, _EF), jnp.float32)
        h = []
        for t in range(_T):
            u = lax.dot_general(xs[t], wup,
                                (((1,), (1,)), ((), ())),
                                preferred_element_type=jnp.float32)
            v = _BETA * v + u
            sp = (v >= _THRESH)
            h.append(sp.astype(jnp.bfloat16))
            v = v - sp.astype(jnp.float32) * _THRESH
        wcol = ws_ref[:, :1]
        v2 = jnp.zeros((tm, _D), jnp.float32)
        for t in range(_T):
            o = lax.dot_general(h[t], wdn, (((1,), (1,)), ((), ())),
                                preferred_element_type=jnp.float32)
            v2 = _BETA * v2 + o
            s2 = (v2 >= _THRESH).astype(jnp.float32)
            v2 = v2 - s2 * _THRESH
            og_refs[t][...] = s2 * wcol


_QR = 16


def _combine_body(og0, og1, og2, og3, s1_hbm, s2_hbm, out_hbm,
                  s1_v, s2_v, ga1, ga2, gb1, gb2,
                  sg1a, sg2a, sg1b, sg2b, soa, sob):
    c = lax.axis_index("c")
    s = lax.axis_index("s")
    wid = s * 2 + c
    base = wid * _CHUNK
    ogs = [og0, og1, og2, og3]
    pltpu.sync_copy(s1_hbm.at[pl.ds(base, _CHUNK)], s1_v)
    pltpu.sync_copy(s2_hbm.at[pl.ds(base, _CHUNK)], s2_v)
    bufs = [(ga1, ga2), (gb1, gb2)]
    gsems = [(sg1a, sg2a), (sg1b, sg2b)]
    osems = [soa, sob]
    nq = _CHUNK // _QR
    nstep = _T * nq
    gcp = [None, None]
    ocp = [None, None]

    def issue(k):
        par = k & 1
        t, q = divmod(k, nq)
        idx1 = s1_v.at[pl.ds(q * _QR, _QR)]
        idx2 = s2_v.at[pl.ds(q * _QR, _QR)]
        c1 = pltpu.async_copy(ogs[t].at[idx1], bufs[par][0], gsems[par][0])
        c2 = pltpu.async_copy(ogs[t].at[idx2], bufs[par][1], gsems[par][1])
        gcp[par] = (c1, c2)

    issue(0)
    for k in range(nstep):
        par = k & 1
        t, q = divmod(k, nq)
        if k + 1 < nstep:
            if ocp[1 - par] is not None:
                ocp[1 - par].wait()
                ocp[1 - par] = None
            issue(k + 1)
        gcp[par][0].wait()
        gcp[par][1].wait()
        g1, g2 = bufs[par]

        def _addrow(r, _):
            for cc in range(_D // 16):
                sl = pl.ds(cc * 16, 16)
                g1[r, sl] = g1[r, sl] + g2[r, sl]
            return 0

        lax.fori_loop(0, _QR, _addrow, 0)
        ocp[par] = pltpu.async_copy(
            g1, out_hbm.at[pl.ds(t * _N + base + q * _QR, _QR)], osems[par])
    for par in range(2):
        if ocp[par] is not None:
            ocp[par].wait()


def _make_combine():
    mesh = plsc.VectorSubcoreMesh(core_axis_name="c", subcore_axis_name="s")
    out_type = jax.ShapeDtypeStruct((_T * _N, _D), jnp.float32)
    scratch = [
        pltpu.VMEM((_CHUNK,), jnp.int32),
        pltpu.VMEM((_CHUNK,), jnp.int32),
        pltpu.VMEM((_QR, _D), jnp.float32),
        pltpu.VMEM((_QR, _D), jnp.float32),
        pltpu.VMEM((_QR, _D), jnp.float32),
        pltpu.VMEM((_QR, _D), jnp.float32),
        pltpu.SemaphoreType.DMA,
        pltpu.SemaphoreType.DMA,
        pltpu.SemaphoreType.DMA,
        pltpu.SemaphoreType.DMA,
        pltpu.SemaphoreType.DMA,
        pltpu.SemaphoreType.DMA,
    ]
    return pl.kernel(_combine_body, mesh=mesh, out_type=out_type,
                     scratch_types=scratch)


def kernel(x, W_up, W_down, expert_bias):
    Tt, Bb, Ss, Dd = x.shape
    N = Bb * Ss
    xf = x.reshape(Tt, N, Dd)
    bias2d = expert_bias.reshape(1, _E)

    TN = 512
    (i1, i2, wrep1, wrep2, p1, p2, cnt, rps, xbf) = pl.pallas_call(
        _routing_kernel,
        grid=(N // TN,),
        in_specs=[
            pl.BlockSpec((Tt, TN, Dd), lambda i: (0, i, 0)),
            pl.BlockSpec((1, _E), lambda i: (0, 0)),
        ],
        out_specs=[
            pl.BlockSpec((1, TN), lambda i: (0, i)),
            pl.BlockSpec((1, TN), lambda i: (0, i)),
            pl.BlockSpec((TN, 128), lambda i: (i, 0)),
            pl.BlockSpec((TN, 128), lambda i: (i, 0)),
            pl.BlockSpec((1, TN), lambda i: (0, i)),
            pl.BlockSpec((1, TN), lambda i: (0, i)),
            pl.BlockSpec((1, _E), lambda i: (0, 0)),
            pl.BlockSpec((1, _E), lambda i: (0, 0)),
            pl.BlockSpec((2, TN, Dd), lambda i: (0, i, 0)),
        ],
        out_shape=[
            jax.ShapeDtypeStruct((1, N), jnp.int32),
            jax.ShapeDtypeStruct((1, N), jnp.int32),
            jax.ShapeDtypeStruct((N, 128), jnp.float32),
            jax.ShapeDtypeStruct((N, 128), jnp.float32),
            jax.ShapeDtypeStruct((1, N), jnp.int32),
            jax.ShapeDtypeStruct((1, N), jnp.int32),
            jax.ShapeDtypeStruct((1, _E), jnp.float32),
            jax.ShapeDtypeStruct((1, _E), jnp.float32),
            jax.ShapeDtypeStruct((2, N, Dd), jnp.int32),
        ],
        scratch_shapes=[pltpu.VMEM((1, _E), jnp.float32)],
    )(xf, bias2d)

    cnt_i = cnt.astype(jnp.int32)
    s1, s2 = pl.pallas_call(
        _slots_kernel,
        grid=(1,),
        in_specs=[
            pl.BlockSpec((1, N), lambda i: (0, 0)),
            pl.BlockSpec((1, N), lambda i: (0, 0)),
            pl.BlockSpec((1, N), lambda i: (0, 0)),
            pl.BlockSpec((1, N), lambda i: (0, 0)),
            pl.BlockSpec(memory_space=pltpu.SMEM),
        ],
        out_specs=[
            pl.BlockSpec((1, N), lambda i: (0, 0)),
            pl.BlockSpec((1, N), lambda i: (0, 0)),
        ],
        out_shape=[
            jax.ShapeDtypeStruct((1, N), jnp.int32),
            jax.ShapeDtypeStruct((1, N), jnp.int32),
        ],
    )(i1, i2, p1, p2, cnt_i)

    tiles = (cnt_i[0] + (_TM - 1)) // _TM
    tcum = jnp.cumsum(tiles)
    g_iota = jnp.arange(_G, dtype=jnp.int32)
    eid = jnp.sum((g_iota[:, None] >= tcum[None, :]).astype(jnp.int32),
                  axis=1)
    eid = jnp.minimum(eid, _E - 1)
    ntiles = tcum[-1:]
    gmap = jnp.minimum(g_iota, ntiles - 1)
    eid = jnp.concatenate([eid, gmap, ntiles])

    dispatch = _make_dispatch()
    s1f = s1.reshape(N)
    s2f = s2.reshape(N)
    xg01, xg23, wslot = dispatch(
        xbf.reshape(2 * N, Dd), s1f, s2f, wrep1, wrep2)

    grid_spec = pltpu.PrefetchScalarGridSpec(
        num_scalar_prefetch=1,
        grid=(_G,),
        in_specs=[
            pl.BlockSpec((_TM, Dd), lambda g, m: (m[_G + g], 0)),
            pl.BlockSpec((_TM, Dd), lambda g, m: (m[_G + g], 0)),
            pl.BlockSpec((_TM, 128), lambda g, m: (m[_G + g], 0)),
            pl.BlockSpec((1, _EF, Dd), lambda g, m: (m[g], 0, 0)),
            pl.BlockSpec((1, Dd, _EF), lambda g, m: (m[g], 0, 0)),
        ],
        out_specs=[
            pl.BlockSpec((_TM, Dd), lambda g, m: (m[_G + g], 0)),
            pl.BlockSpec((_TM, Dd), lambda g, m: (m[_G + g], 0)),
            pl.BlockSpec((_TM, Dd), lambda g, m: (m[_G + g], 0)),
            pl.BlockSpec((_TM, Dd), lambda g, m: (m[_G + g], 0)),
        ],
    )
    og0, og1, og2, og3 = pl.pallas_call(
        _group_kernel,
        grid_spec=grid_spec,
        out_shape=[jax.ShapeDtypeStruct((_P, Dd), jnp.float32)
                   for _ in range(_T)],
    )(eid, xg01, xg23, wslot,
      W_up.astype(jnp.bfloat16), W_down.astype(jnp.bfloat16))

    combine = _make_combine()
    out_flat = combine(og0, og1, og2, og3, s1f, s2f)

    ef_frac = cnt[0] / (N * _TOPK)
    rp = rps[0] / N
    lb = _E * jnp.sum(ef_frac * rp)
    return out_flat.reshape(Tt, Bb, Ss, Dd), lb


# speedup vs baseline: 1.1236x; 1.0027x over previous
import jax
import jax.numpy as jnp
from jax import lax
from jax.experimental import pallas as pl
from jax.experimental.pallas import tpu as pltpu
from jax.experimental.pallas import tpu_sc as plsc

_T, _B, _S, _D = 4, 1, 2048, 1024
_N = _B * _S
_E = 8
_TOPK = 2
_NC = 64
_DFF = 4096
_EF = _DFF // _E
_CPE = _NC // _E
_BETA = 0.5
_THRESH = 1.0

_TM = 256
_G = (_TOPK * _N) // _TM + _E
_P = _G * _TM

_NW = 32
_CHUNK = _N // _NW


def _routing_kernel(x_ref, bias_ref, i1_ref, i2_ref, w1_ref, w2_ref,
                    p1_ref, p2_ref, cnt_ref, rps_ref, xbf_ref, base_ref):
    i = pl.program_id(0)
    tn = x_ref.shape[1]
    d = x_ref.shape[2]

    @pl.when(i == 0)
    def _():
        base_ref[...] = jnp.zeros_like(base_ref)

    v = jnp.zeros((tn, d), jnp.float32)
    fr = jnp.zeros((tn, d), jnp.float32)
    for t in range(_T):
        v = _BETA * v + x_ref[t]
        s = (v >= _THRESH).astype(jnp.float32)
        fr = fr + s
        v = v - s * _THRESH
    dmask = ((jnp.right_shift(lax.broadcasted_iota(jnp.int32, (d, _E), 0), 3)
              & (_E - 1))
             == lax.broadcasted_iota(jnp.int32, (d, _E), 1)).astype(jnp.float32)
    es = lax.dot_general(fr, dmask, (((1,), (0,)), ((), ())),
                         preferred_element_type=jnp.float32)
    es = es * (1.0 / (_T * (_D // _NC) * _CPE))
    es = es + bias_ref[0][None, :]
    eidx = lax.broadcasted_iota(jnp.int32, (tn, _E), 1)
    m1 = jnp.max(es, axis=1)
    i1 = jnp.min(jnp.where(es == m1[:, None], eidx, _E), axis=1)
    masked = jnp.where(eidx == i1[:, None], -jnp.inf, es)
    m2 = jnp.max(masked, axis=1)
    i2 = jnp.min(jnp.where(masked == m2[:, None], eidx, _E), axis=1)
    eb = jnp.exp(m2 - m1)
    w1 = 1.0 / (1.0 + eb)
    w2 = eb / (1.0 + eb)
    oh1 = (eidx == i1[:, None]).astype(jnp.float32)
    oh2 = (eidx == i2[:, None]).astype(jnp.float32)
    assigned = oh1 + oh2
    r_iota = lax.broadcasted_iota(jnp.int32, (tn, tn), 0)
    c_iota = lax.broadcasted_iota(jnp.int32, (tn, tn), 1)
    tril = (r_iota > c_iota).astype(jnp.float32)
    ranks = lax.dot_general(tril, assigned, (((1,), (0,)), ((), ())),
                            preferred_element_type=jnp.float32)
    base = base_ref[0][None, :]
    pos = ranks + base
    p1 = jnp.sum(oh1 * pos, axis=1)
    p2 = jnp.sum(oh2 * pos, axis=1)
    base_ref[...] = base + jnp.sum(assigned, axis=0, keepdims=True)

    mask_hi = jnp.int32(-65536)
    xbits = []
    for t in range(_T):
        rt = x_ref[t].astype(jnp.bfloat16).astype(jnp.float32)
        xbits.append(lax.bitcast_convert_type(rt, jnp.int32))
    w01 = (xbits[1] & mask_hi) | lax.shift_right_logical(xbits[0], 16)
    w23 = (xbits[3] & mask_hi) | lax.shift_right_logical(xbits[2], 16)
    xbf_ref[0] = w01
    xbf_ref[1] = w23
    i1_ref[...] = i1[None, :]
    i2_ref[...] = i2[None, :]
    w1_ref[...] = jnp.broadcast_to(w1[:, None], (tn, 128))
    w2_ref[...] = jnp.broadcast_to(w2[:, None], (tn, 128))
    p1_ref[...] = p1[None, :].astype(jnp.int32)
    p2_ref[...] = p2[None, :].astype(jnp.int32)

    cnt = jnp.sum(assigned, axis=0)
    ex = jnp.exp(es - m1[:, None])
    rp = ex / jnp.sum(ex, axis=1, keepdims=True)
    rps = jnp.sum(rp, axis=0)

    @pl.when(i == 0)
    def _():
        cnt_ref[...] = cnt[None, :]
        rps_ref[...] = rps[None, :]

    @pl.when(i > 0)
    def _():
        cnt_ref[...] += cnt[None, :]
        rps_ref[...] += rps[None, :]


def _slots_kernel(i1_ref, i2_ref, p1_ref, p2_ref, cnt_ref, s1_ref, s2_ref):
    i1v = i1_ref[...]
    i2v = i2_ref[...]
    off1 = jnp.zeros_like(i1v)
    off2 = jnp.zeros_like(i2v)
    running = 0
    for e in range(_E):
        off1 = jnp.where(i1v == e, running, off1)
        off2 = jnp.where(i2v == e, running, off2)
        ce = cnt_ref[0, e]
        running = running + ((ce + (_TM - 1)) // _TM) * _TM
    s1_ref[...] = p1_ref[...] + off1
    s2_ref[...] = p2_ref[...] + off2


_SUB = 32


def _dispatch_body(x_hbm, s1_hbm, s2_hbm, w1_hbm, w2_hbm,
                   xg01, xg23, ws_hbm,
                   s1_v, s2_v, s1h, s2h, wrow_v, xa, xb,
                   semw, sia, sib, ssa1, ssa2, ssb1, ssb2):
    c = lax.axis_index("c")
    s = lax.axis_index("s")
    wid = s * 2 + c
    base = wid * _CHUNK
    pltpu.sync_copy(s1_hbm.at[pl.ds(base, _CHUNK)], s1_v)
    pltpu.sync_copy(s2_hbm.at[pl.ds(base, _CHUNK)], s2_v)
    for hh in range(_CHUNK // _SUB):
        pltpu.sync_copy(s1_hbm.at[pl.ds(base + hh * _SUB, _SUB)], s1h.at[hh])
        pltpu.sync_copy(s2_hbm.at[pl.ds(base + hh * _SUB, _SUB)], s2h.at[hh])
    pltpu.sync_copy(w1_hbm.at[pl.ds(base, _CHUNK)], wrow_v)
    cpw = pltpu.async_copy(wrow_v, ws_hbm.at[s1_v], semw)
    cpw.wait()
    pltpu.sync_copy(w2_hbm.at[pl.ds(base, _CHUNK)], wrow_v)
    cpw = pltpu.async_copy(wrow_v, ws_hbm.at[s2_v], semw)
    cpw.wait()
    xgs = [xg01, xg23]
    bufs = [xa, xb]
    isems = [sia, sib]
    ssems = [(ssa1, ssa2), (ssb1, ssb2)]
    nh = _CHUNK // _SUB
    nstep = 2 * nh
    ins = [None, None]
    scs = [None, None]

    def issue_in(k):
        par = k & 1
        t, hh = divmod(k, nh)
        ins[par] = pltpu.async_copy(
            x_hbm.at[pl.ds(t * _N + base + hh * _SUB, _SUB)],
            bufs[par], isems[par])

    issue_in(0)
    for k in range(nstep):
        par = k & 1
        t, hh = divmod(k, nh)
        if k + 1 < nstep:
            if scs[1 - par] is not None:
                scs[1 - par][0].wait()
                scs[1 - par][1].wait()
                scs[1 - par] = None
            issue_in(k + 1)
        ins[par].wait()
        c1 = pltpu.async_copy(bufs[par], xgs[t].at[s1h.at[hh]],
                              ssems[par][0])
        c2 = pltpu.async_copy(bufs[par], xgs[t].at[s2h.at[hh]],
                              ssems[par][1])
        scs[par] = (c1, c2)
    for par in range(2):
        if scs[par] is not None:
            scs[par][0].wait()
            scs[par][1].wait()


def _make_dispatch():
    mesh = plsc.VectorSubcoreMesh(core_axis_name="c", subcore_axis_name="s")
    out_type = (
        [jax.ShapeDtypeStruct((_P, _D), jnp.int32) for _ in range(2)]
        + [jax.ShapeDtypeStruct((_P, 128), jnp.float32)]
    )
    scratch = [
        pltpu.VMEM((_CHUNK,), jnp.int32),
        pltpu.VMEM((_CHUNK,), jnp.int32),
        pltpu.VMEM((_CHUNK // _SUB, _SUB), jnp.int32),
        pltpu.VMEM((_CHUNK // _SUB, _SUB), jnp.int32),
        pltpu.VMEM((_CHUNK, 128), jnp.float32),
        pltpu.VMEM((_SUB, _D), jnp.int32),
        pltpu.VMEM((_SUB, _D), jnp.int32),
        pltpu.SemaphoreType.DMA,
        pltpu.SemaphoreType.DMA,
        pltpu.SemaphoreType.DMA,
        pltpu.SemaphoreType.DMA,
        pltpu.SemaphoreType.DMA,
        pltpu.SemaphoreType.DMA,
        pltpu.SemaphoreType.DMA,
    ]
    return pl.kernel(_dispatch_body, mesh=mesh, out_type=out_type,
                     scratch_types=scratch)


def _group_kernel(eid_ref, xg01_ref, xg23_ref,
                  ws_ref, wup_ref, wdn_ref,
                  og0_ref, og1_ref, og2_ref, og3_ref):
    g = pl.program_id(0)
    ntiles = eid_ref[2 * _G]

    @pl.when(g < ntiles)
    def _():
        og_refs = [og0_ref, og1_ref, og2_ref, og3_ref]
        tm = xg01_ref.shape[0]
        mask_hi = jnp.int32(-65536)
        xs = []
        for wref in (xg01_ref, xg23_ref):
            w = wref[...]
            xs.append(lax.bitcast_convert_type(
                lax.shift_left(w, 16), jnp.float32))
            xs.append(lax.bitcast_convert_type(w & mask_hi, jnp.float32))
        wup = wup_ref[0]
        wdn = wdn_ref[0]
        v = jnp.zeros((tm, _EF), jnp.float32)
        h = []
        for t in range(_T):
            u = lax.dot_general(xs[t], wup,
                                (((1,), (1,)), ((), ())),
                                preferred_element_type=jnp.float32)
            v = _BETA * v + u
            sp = (v >= _THRESH)
            h.append(sp.astype(jnp.bfloat16))
            v = v - sp.astype(jnp.float32) * _THRESH
        wcol = ws_ref[:, :1]
        v2 = jnp.zeros((tm, _D), jnp.float32)
        for t in range(_T):
            o = lax.dot_general(h[t], wdn, (((1,), (1,)), ((), ())),
                                preferred_element_type=jnp.float32)
            v2 = _BETA * v2 + o
            s2 = (v2 >= _THRESH).astype(jnp.float32)
            v2 = v2 - s2 * _THRESH
            og_refs[t][...] = s2 * wcol


_QR = 16


def _combine_body(og0, og1, og2, og3, s1_hbm, s2_hbm, out_hbm,
                  s1_v, s2_v, ga1, ga2, gb1, gb2,
                  sg1a, sg2a, sg1b, sg2b, soa, sob):
    c = lax.axis_index("c")
    s = lax.axis_index("s")
    wid = s * 2 + c
    base = wid * _CHUNK
    ogs = [og0, og1, og2, og3]
    pltpu.sync_copy(s1_hbm.at[pl.ds(base, _CHUNK)], s1_v)
    pltpu.sync_copy(s2_hbm.at[pl.ds(base, _CHUNK)], s2_v)
    bufs = [(ga1, ga2), (gb1, gb2)]
    gsems = [(sg1a, sg2a), (sg1b, sg2b)]
    osems = [soa, sob]
    nq = _CHUNK // _QR
    nstep = _T * nq
    gcp = [None, None]
    ocp = [None, None]

    def issue(k):
        par = k & 1
        t, q = divmod(k, nq)
        idx1 = s1_v.at[pl.ds(q * _QR, _QR)]
        idx2 = s2_v.at[pl.ds(q * _QR, _QR)]
        c1 = pltpu.async_copy(ogs[t].at[idx1], bufs[par][0], gsems[par][0])
        c2 = pltpu.async_copy(ogs[t].at[idx2], bufs[par][1], gsems[par][1])
        gcp[par] = (c1, c2)

    issue(0)
    for k in range(nstep):
        par = k & 1
        t, q = divmod(k, nq)
        if k + 1 < nstep:
            if ocp[1 - par] is not None:
                ocp[1 - par].wait()
                ocp[1 - par] = None
            issue(k + 1)
        gcp[par][0].wait()
        gcp[par][1].wait()
        g1, g2 = bufs[par]

        def _addrow(r, _):
            for cc in range(_D // 16):
                sl = pl.ds(cc * 16, 16)
                g1[r, sl] = g1[r, sl] + g2[r, sl]
            return 0

        lax.fori_loop(0, _QR, _addrow, 0)
        ocp[par] = pltpu.async_copy(
            g1, out_hbm.at[pl.ds(t * _N + base + q * _QR, _QR)], osems[par])
    for par in range(2):
        if ocp[par] is not None:
            ocp[par].wait()


def _make_combine():
    mesh = plsc.VectorSubcoreMesh(core_axis_name="c", subcore_axis_name="s")
    out_type = jax.ShapeDtypeStruct((_T * _N, _D), jnp.float32)
    scratch = [
        pltpu.VMEM((_CHUNK,), jnp.int32),
        pltpu.VMEM((_CHUNK,), jnp.int32),
        pltpu.VMEM((_QR, _D), jnp.float32),
        pltpu.VMEM((_QR, _D), jnp.float32),
        pltpu.VMEM((_QR, _D), jnp.float32),
        pltpu.VMEM((_QR, _D), jnp.float32),
        pltpu.SemaphoreType.DMA,
        pltpu.SemaphoreType.DMA,
        pltpu.SemaphoreType.DMA,
        pltpu.SemaphoreType.DMA,
        pltpu.SemaphoreType.DMA,
        pltpu.SemaphoreType.DMA,
    ]
    return pl.kernel(_combine_body, mesh=mesh, out_type=out_type,
                     scratch_types=scratch)


def kernel(x, W_up, W_down, expert_bias):
    Tt, Bb, Ss, Dd = x.shape
    N = Bb * Ss
    xf = x.reshape(Tt, N, Dd)
    bias2d = expert_bias.reshape(1, _E)

    TN = 512
    (i1, i2, wrep1, wrep2, p1, p2, cnt, rps, xbf) = pl.pallas_call(
        _routing_kernel,
        grid=(N // TN,),
        in_specs=[
            pl.BlockSpec((Tt, TN, Dd), lambda i: (0, i, 0)),
            pl.BlockSpec((1, _E), lambda i: (0, 0)),
        ],
        out_specs=[
            pl.BlockSpec((1, TN), lambda i: (0, i)),
            pl.BlockSpec((1, TN), lambda i: (0, i)),
            pl.BlockSpec((TN, 128), lambda i: (i, 0)),
            pl.BlockSpec((TN, 128), lambda i: (i, 0)),
            pl.BlockSpec((1, TN), lambda i: (0, i)),
            pl.BlockSpec((1, TN), lambda i: (0, i)),
            pl.BlockSpec((1, _E), lambda i: (0, 0)),
            pl.BlockSpec((1, _E), lambda i: (0, 0)),
            pl.BlockSpec((2, TN, Dd), lambda i: (0, i, 0)),
        ],
        out_shape=[
            jax.ShapeDtypeStruct((1, N), jnp.int32),
            jax.ShapeDtypeStruct((1, N), jnp.int32),
            jax.ShapeDtypeStruct((N, 128), jnp.float32),
            jax.ShapeDtypeStruct((N, 128), jnp.float32),
            jax.ShapeDtypeStruct((1, N), jnp.int32),
            jax.ShapeDtypeStruct((1, N), jnp.int32),
            jax.ShapeDtypeStruct((1, _E), jnp.float32),
            jax.ShapeDtypeStruct((1, _E), jnp.float32),
            jax.ShapeDtypeStruct((2, N, Dd), jnp.int32),
        ],
        scratch_shapes=[pltpu.VMEM((1, _E), jnp.float32)],
    )(xf, bias2d)

    cnt_i = cnt.astype(jnp.int32)
    s1, s2 = pl.pallas_call(
        _slots_kernel,
        grid=(1,),
        in_specs=[
            pl.BlockSpec((1, N), lambda i: (0, 0)),
            pl.BlockSpec((1, N), lambda i: (0, 0)),
            pl.BlockSpec((1, N), lambda i: (0, 0)),
            pl.BlockSpec((1, N), lambda i: (0, 0)),
            pl.BlockSpec(memory_space=pltpu.SMEM),
        ],
        out_specs=[
            pl.BlockSpec((1, N), lambda i: (0, 0)),
            pl.BlockSpec((1, N), lambda i: (0, 0)),
        ],
        out_shape=[
            jax.ShapeDtypeStruct((1, N), jnp.int32),
            jax.ShapeDtypeStruct((1, N), jnp.int32),
        ],
    )(i1, i2, p1, p2, cnt_i)

    tiles = (cnt_i[0] + (_TM - 1)) // _TM
    tcum = jnp.cumsum(tiles)
    g_iota = jnp.arange(_G, dtype=jnp.int32)
    eid = jnp.sum((g_iota[:, None] >= tcum[None, :]).astype(jnp.int32),
                  axis=1)
    eid = jnp.minimum(eid, _E - 1)
    ntiles = tcum[-1:]
    gmap = jnp.minimum(g_iota, ntiles - 1)
    eid = jnp.concatenate([eid, gmap, ntiles])

    dispatch = _make_dispatch()
    s1f = s1.reshape(N)
    s2f = s2.reshape(N)
    xg01, xg23, wslot = dispatch(
        xbf.reshape(2 * N, Dd), s1f, s2f, wrep1, wrep2)

    grid_spec = pltpu.PrefetchScalarGridSpec(
        num_scalar_prefetch=1,
        grid=(_G,),
        in_specs=[
            pl.BlockSpec((_TM, Dd), lambda g, m: (m[_G + g], 0)),
            pl.BlockSpec((_TM, Dd), lambda g, m: (m[_G + g], 0)),
            pl.BlockSpec((_TM, 128), lambda g, m: (m[_G + g], 0)),
            pl.BlockSpec((1, _EF, Dd), lambda g, m: (m[g], 0, 0)),
            pl.BlockSpec((1, Dd, _EF), lambda g, m: (m[g], 0, 0)),
        ],
        out_specs=[
            pl.BlockSpec((_TM, Dd), lambda g, m: (m[_G + g], 0)),
            pl.BlockSpec((_TM, Dd), lambda g, m: (m[_G + g], 0)),
            pl.BlockSpec((_TM, Dd), lambda g, m: (m[_G + g], 0)),
            pl.BlockSpec((_TM, Dd), lambda g, m: (m[_G + g], 0)),
        ],
    )
    og0, og1, og2, og3 = pl.pallas_call(
        _group_kernel,
        grid_spec=grid_spec,
        out_shape=[jax.ShapeDtypeStruct((_P, Dd), jnp.float32)
                   for _ in range(_T)],
    )(eid, xg01, xg23, wslot,
      W_up.astype(jnp.bfloat16), W_down.astype(jnp.bfloat16))

    combine = _make_combine()
    out_flat = combine(og0, og1, og2, og3, s1f, s2f)

    ef_frac = cnt[0] / (N * _TOPK)
    rp = rps[0] / N
    lb = _E * jnp.sum(ef_frac * rp)
    return out_flat.reshape(Tt, Bb, Ss, Dd), lb
